# baseline (speedup 1.0000x reference)
import functools

import jax
import jax.numpy as jnp
from jax import lax
from jax.experimental import pallas as pl
from jax.experimental.pallas import tpu as pltpu
from jax.experimental.pallas import tpu_sc as plsc

N = 10000
NP = 10240
L = 128
TWO_L = 256
FOUR_L = 512
T = 4
E = 80000
K = 128
NUM_CHUNKS = E // K
NW = 32
ROWS_PER_TILE = NP // 16
B = 2000
GRID = N // B


_mesh = plsc.VectorSubcoreMesh(core_axis_name="c", subcore_axis_name="s")


CPW = 20
CPW_PAD = 24


@functools.partial(
    pl.kernel,
    out_type=[jax.ShapeDtypeStruct((2, NP, L), jnp.float32) for _ in range(T)],
    mesh=_mesh,
    scratch_types=[
        pltpu.VMEM((CPW_PAD, K), jnp.int32),
        pltpu.VMEM((CPW_PAD, K), jnp.int32),
        pltpu.VMEM((CPW_PAD, K), jnp.float32),
        pltpu.VMEM((K, L), jnp.float32),
        pltpu.VMEM((K, L), jnp.float32),
        pltpu.VMEM((32, L), jnp.float32),
        pltpu.VMEM_SHARED((NP, L), jnp.float32),
        pltpu.SemaphoreType.DMA,
        pltpu.SemaphoreType.DMA,
        pltpu.SemaphoreType.DMA,
    ],
)
def _spmm_all_types(ch0, ch1, ch2, ch3, esrc, edst, ew, o0, o1, o2, o3,
                    sidx, didx, wv, rows0, rows1, zbuf, acc,
                    sem0, sem1, semi):
    cid = lax.axis_index("c")
    sid = lax.axis_index("s")
    wid = sid * 2 + cid
    row0 = sid * ROWS_PER_TILE

    z16 = jnp.zeros((16,), jnp.float32)

    def _zrow(r, carry):
        for c in range(8):
            zbuf[r, pl.ds(16 * c, 16)] = z16
        return carry

    lax.fori_loop(0, 32, _zrow, 0)

    chs = [ch0, ch1, ch2, ch3]
    outs = [o0, o1, o2, o3]

    for t in range(T):
        rb = (t * NW + wid) * CPW_PAD
        pltpu.async_copy(esrc.at[pl.ds(rb, CPW_PAD)], sidx, semi)
        pltpu.async_copy(edst.at[pl.ds(rb, CPW_PAD)], didx, semi)
        pltpu.async_copy(ew.at[pl.ds(rb, CPW_PAD)], wv, semi)
        def _zcopy(b, carry):
            pltpu.sync_copy(zbuf, acc.at[pl.ds(row0 + 32 * b, 32)])
            return carry

        lax.fori_loop(0, ROWS_PER_TILE // 32, _zcopy, 0)
        pltpu.make_async_copy(esrc.at[pl.ds(rb, CPW_PAD)], sidx, semi).wait()
        pltpu.make_async_copy(edst.at[pl.ds(rb, CPW_PAD)], didx, semi).wait()
        pltpu.make_async_copy(ew.at[pl.ds(rb, CPW_PAD)], wv, semi).wait()
        plsc.subcore_barrier()

        ch = chs[t]

        def _gather(c, buf, sem):
            return pltpu.async_copy(ch.at[sidx.at[c]], buf, sem)

        def _wait(buf, sem):
            pltpu.make_async_copy(ch.at[sidx.at[0]], buf, sem).wait()

        def _scale_buf(c, buf):
            def _scale(g, c2):
                w16 = wv[c, pl.ds(g * 16, 16)]
                for ll in range(16):
                    j = g * 16 + ll
                    wsp = w16[ll]
                    for cc in range(8):
                        sl = pl.ds(16 * cc, 16)
                        buf[j, sl] = buf[j, sl] * wsp
                return c2

            lax.fori_loop(0, K // 16, _scale, 0)

        def _scatter(c, buf, sem):
            pltpu.async_copy(buf, acc.at[didx.at[c]], sem, add=True)

        def _wait_scatter(buf, sem):
            pltpu.make_async_copy(buf, acc.at[didx.at[0]], sem).wait()

        _gather(0, rows0, sem0)
        _gather(1, rows1, sem1)

        def _pair(p, carry):
            _wait(rows0, sem0)
            _scale_buf(2 * p, rows0)
            _scatter(2 * p, rows0, sem0)

            _wait(rows1, sem1)
            _scale_buf(2 * p + 1, rows1)
            _scatter(2 * p + 1, rows1, sem1)

            _wait_scatter(rows0, sem0)

            @pl.when(p < CPW // 2 - 1)
            def _():
                _gather(2 * p + 2, rows0, sem0)

            _wait_scatter(rows1, sem1)

            @pl.when(p < CPW // 2 - 1)
            def _():
                _gather(2 * p + 3, rows1, sem1)

            return carry

        lax.fori_loop(0, CPW // 2, _pair, 0)
        plsc.subcore_barrier()

        for b5 in range(5):
            sl = pl.ds(row0 + K * b5, K)
            pltpu.sync_copy(acc.at[sl], outs[t].at[cid, sl])
        plsc.subcore_barrier()


def _full(shape):
    return pl.BlockSpec(shape, lambda i, _s=shape: tuple(0 for _ in _s))


def _rowblk(w):
    return pl.BlockSpec((B, w), lambda i: (i, 0))


def _pre_body(nf, ae, wav, w1, b1, cw, cb, cur_o, c0, c1, c2, c3):
    ws = jnp.sum(wav[...], axis=0, keepdims=True)
    left = jnp.tanh(
        jnp.dot(nf[...], w1[...], preferred_element_type=jnp.float32)
        + b1[...] + ae[...])
    right = jnp.broadcast_to(jnp.tanh(ws), (B, L))
    cur = jnp.concatenate([left, right], axis=1)
    cur_o[...] = cur
    cf = jnp.dot(cur, cw[...], preferred_element_type=jnp.float32) + cb[...]
    c0[...] = cf[:, 0:L]
    c1[...] = cf[:, L:2 * L]
    c2[...] = cf[:, 2 * L:3 * L]
    c3[...] = cf[:, 3 * L:4 * L]


_pre_call = pl.pallas_call(
    _pre_body,
    grid=(GRID,),
    in_specs=[_rowblk(L), _rowblk(L), _full((512, L)), _full((L, L)),
              _full((1, L)), _full((TWO_L, FOUR_L)), _full((1, FOUR_L))],
    out_specs=[_rowblk(TWO_L)] + [_rowblk(L)] * 4,
    out_shape=[jax.ShapeDtypeStruct((N, TWO_L), jnp.float32)]
    + [jax.ShapeDtypeStruct((N, L), jnp.float32)] * 4,
)


def _merge_core(ps, cur, mw, mb):
    mwv = mw[...]
    s = jnp.zeros((B, TWO_L), jnp.float32)
    for i in range(4):
        pv = ps[i][...]
        m = jnp.tanh(pv[0] + pv[1])
        s = s + jnp.dot(m, mwv[i * L:(i + 1) * L, :],
                        preferred_element_type=jnp.float32)
    return jnp.tanh(s + mb[...] + cur[...])


def _merge_conv_body(p0, p1, p2, p3, cur, mw, mb, cw, cb,
                     cur_o, c0, c1, c2, c3):
    cur2 = _merge_core([p0, p1, p2, p3], cur, mw, mb)
    cur_o[...] = cur2
    cf = jnp.dot(cur2, cw[...], preferred_element_type=jnp.float32) + cb[...]
    c0[...] = cf[:, 0:L]
    c1[...] = cf[:, L:2 * L]
    c2[...] = cf[:, 2 * L:3 * L]
    c3[...] = cf[:, 3 * L:4 * L]


def _merge_final_body(p0, p1, p2, p3, cur, mw, mb, cur_o):
    cur_o[...] = _merge_core([p0, p1, p2, p3], cur, mw, mb)


def _pblk():
    return pl.BlockSpec((2, B, L), lambda i: (0, i, 0))


_merge_conv_call = pl.pallas_call(
    _merge_conv_body,
    grid=(GRID,),
    in_specs=[_pblk(), _pblk(), _pblk(), _pblk(),
              _rowblk(TWO_L), _full((FOUR_L, TWO_L)), _full((1, TWO_L)),
              _full((TWO_L, FOUR_L)), _full((1, FOUR_L))],
    out_specs=[_rowblk(TWO_L)] + [_rowblk(L)] * 4,
    out_shape=[jax.ShapeDtypeStruct((N, TWO_L), jnp.float32)]
    + [jax.ShapeDtypeStruct((N, L), jnp.float32)] * 4,
)

_merge_final_call = pl.pallas_call(
    _merge_final_body,
    grid=(GRID,),
    in_specs=[_pblk(), _pblk(), _pblk(), _pblk(),
              _rowblk(TWO_L), _full((FOUR_L, TWO_L)), _full((1, TWO_L))],
    out_specs=_rowblk(TWO_L),
    out_shape=jax.ShapeDtypeStruct((N, TWO_L), jnp.float32),
)


def kernel(node_feat, edge_index, edge_weight, all_embedding, wave_embedding,
           w_n2l_W, w_n2l_b, conv_W, conv_b, merge_W, merge_b):
    cur, c0, c1, c2, c3 = _pre_call(
        node_feat, all_embedding, wave_embedding, w_n2l_W,
        w_n2l_b.reshape(1, L), conv_W[0], conv_b[0].reshape(1, FOUR_L))
    def _blocks(x, pad_vals):
        pad = jnp.broadcast_to(pad_vals, (T, NW * CPW * K - E))
        x = jnp.concatenate([x, pad.astype(x.dtype)], axis=1)
        x = x.reshape(T, CPW, NW, K).transpose(0, 2, 1, 3)
        x = jnp.pad(x, ((0, 0), (0, 0), (0, CPW_PAD - CPW), (0, 0)))
        return x.reshape(T * NW * CPW_PAD, K)

    n_pad = NW * CPW * K - E
    idx_pad = jnp.arange(n_pad, dtype=jnp.int32)
    esrc = _blocks(edge_index[:, 0, :], idx_pad % N)
    edst = _blocks(edge_index[:, 1, :], N + idx_pad % (NP - N))
    ew = _blocks(edge_weight, jnp.zeros((n_pad,), jnp.float32))
    for lv in range(3):
        p0, p1, p2, p3 = _spmm_all_types(c0, c1, c2, c3, esrc, edst, ew)
        if lv < 2:
            cur, c0, c1, c2, c3 = _merge_conv_call(
                p0, p1, p2, p3, cur, merge_W[lv], merge_b[lv].reshape(1, TWO_L),
                conv_W[lv + 1], conv_b[lv + 1].reshape(1, FOUR_L))
        else:
            cur = _merge_final_call(
                p0, p1, p2, p3, cur, merge_W[2], merge_b[2].reshape(1, TWO_L))
    return cur


# speedup vs baseline: 1.1019x; 1.1019x over previous
import functools

import jax
import jax.numpy as jnp
from jax import lax
from jax.experimental import pallas as pl
from jax.experimental.pallas import tpu as pltpu
from jax.experimental.pallas import tpu_sc as plsc

N = 10000
NP = 10240
L = 128
TWO_L = 256
FOUR_L = 512
T = 4
E = 80000
K = 128
NUM_CHUNKS = E // K
NW = 32
ROWS_PER_TILE = NP // 16
B = 2000
GRID = N // B


_mesh = plsc.VectorSubcoreMesh(core_axis_name="c", subcore_axis_name="s")


CPW = 20
CPW_PAD = 24


@functools.partial(
    pl.kernel,
    out_type=[jax.ShapeDtypeStruct((2, NP, L), jnp.float32) for _ in range(T)],
    mesh=_mesh,
    scratch_types=[
        pltpu.VMEM((CPW_PAD, K), jnp.int32),
        pltpu.VMEM((CPW_PAD, K), jnp.int32),
        pltpu.VMEM((CPW_PAD, K), jnp.float32),
        pltpu.VMEM((K, L), jnp.float32),
        pltpu.VMEM((K, L), jnp.float32),
        pltpu.VMEM((32, L), jnp.float32),
        pltpu.VMEM_SHARED((NP, L), jnp.float32),
        pltpu.SemaphoreType.DMA,
        pltpu.SemaphoreType.DMA,
        pltpu.SemaphoreType.DMA,
    ],
)
def _spmm_all_types(ch0, ch1, ch2, ch3, esrc, edst, ew, o0, o1, o2, o3,
                    sidx, didx, wv, rows0, rows1, zbuf, acc,
                    sem0, sem1, semi):
    cid = lax.axis_index("c")
    sid = lax.axis_index("s")
    wid = sid * 2 + cid
    row0 = sid * ROWS_PER_TILE

    z16 = jnp.zeros((16,), jnp.float32)

    def _zrow(r, carry):
        for c in range(8):
            zbuf[r, pl.ds(16 * c, 16)] = z16
        return carry

    lax.fori_loop(0, 32, _zrow, 0)

    chs = [ch0, ch1, ch2, ch3]
    outs = [o0, o1, o2, o3]

    for t in range(T):
        rb = (t * NW + wid) * CPW_PAD
        pltpu.async_copy(esrc.at[pl.ds(rb, CPW_PAD)], sidx, semi)
        pltpu.async_copy(edst.at[pl.ds(rb, CPW_PAD)], didx, semi)
        pltpu.async_copy(ew.at[pl.ds(rb, CPW_PAD)], wv, semi)
        pltpu.make_async_copy(esrc.at[pl.ds(rb, CPW_PAD)], sidx, semi).wait()
        pltpu.make_async_copy(edst.at[pl.ds(rb, CPW_PAD)], didx, semi).wait()
        pltpu.make_async_copy(ew.at[pl.ds(rb, CPW_PAD)], wv, semi).wait()

        ch = chs[t]

        def _gather(c, buf, sem):
            return pltpu.async_copy(ch.at[sidx.at[c]], buf, sem)

        def _wait(buf, sem):
            pltpu.make_async_copy(ch.at[sidx.at[0]], buf, sem).wait()

        _gather(0, rows0, sem0)
        _gather(1, rows1, sem1)

        def _zcopy(b, carry):
            pltpu.async_copy(zbuf, acc.at[pl.ds(row0 + 32 * b, 32)], semi)
            return carry

        lax.fori_loop(0, ROWS_PER_TILE // 32, _zcopy, 0)

        def _zwait(b, carry):
            pltpu.make_async_copy(zbuf, acc.at[pl.ds(row0, 32)], semi).wait()
            return carry

        lax.fori_loop(0, ROWS_PER_TILE // 32, _zwait, 0)
        plsc.subcore_barrier()

        def _scale_buf(c, buf):
            def _scale(g, c2):
                w16 = wv[c, pl.ds(g * 16, 16)]
                for ll in range(16):
                    j = g * 16 + ll
                    wsp = w16[ll]
                    for cc in range(8):
                        sl = pl.ds(16 * cc, 16)
                        buf[j, sl] = buf[j, sl] * wsp
                return c2

            lax.fori_loop(0, K // 16, _scale, 0)

        def _pair(p, carry):
            _wait(rows0, sem0)
            _scale_buf(2 * p, rows0)
            pltpu.sync_copy(rows0, acc.at[didx.at[2 * p]], add=True)

            @pl.when(p < CPW // 2 - 1)
            def _():
                _gather(2 * p + 2, rows0, sem0)

            _wait(rows1, sem1)
            _scale_buf(2 * p + 1, rows1)
            pltpu.sync_copy(rows1, acc.at[didx.at[2 * p + 1]], add=True)

            @pl.when(p < CPW // 2 - 1)
            def _():
                _gather(2 * p + 3, rows1, sem1)

            return carry

        lax.fori_loop(0, CPW // 2, _pair, 0)
        plsc.subcore_barrier()

        for b5 in range(5):
            sl = pl.ds(row0 + K * b5, K)
            pltpu.async_copy(acc.at[sl], outs[t].at[cid, sl], semi)
        for b5 in range(5):
            sl = pl.ds(row0 + K * b5, K)
            pltpu.make_async_copy(acc.at[sl], outs[t].at[cid, sl], semi).wait()


def _full(shape):
    return pl.BlockSpec(shape, lambda i, _s=shape: tuple(0 for _ in _s))


def _rowblk(w):
    return pl.BlockSpec((B, w), lambda i: (i, 0))


def _pre_body(nf, ae, wav, w1, b1, cw, cb, cur_o, c0, c1, c2, c3):
    ws = jnp.sum(wav[...], axis=0, keepdims=True)
    left = jnp.tanh(
        jnp.dot(nf[...], w1[...], preferred_element_type=jnp.float32)
        + b1[...] + ae[...])
    right = jnp.broadcast_to(jnp.tanh(ws), (B, L))
    cur = jnp.concatenate([left, right], axis=1)
    cur_o[...] = cur
    cf = jnp.dot(cur, cw[...], preferred_element_type=jnp.float32) + cb[...]
    c0[...] = cf[:, 0:L]
    c1[...] = cf[:, L:2 * L]
    c2[...] = cf[:, 2 * L:3 * L]
    c3[...] = cf[:, 3 * L:4 * L]


_pre_call = pl.pallas_call(
    _pre_body,
    grid=(GRID,),
    in_specs=[_rowblk(L), _rowblk(L), _full((512, L)), _full((L, L)),
              _full((1, L)), _full((TWO_L, FOUR_L)), _full((1, FOUR_L))],
    out_specs=[_rowblk(TWO_L)] + [_rowblk(L)] * 4,
    out_shape=[jax.ShapeDtypeStruct((N, TWO_L), jnp.float32)]
    + [jax.ShapeDtypeStruct((N, L), jnp.float32)] * 4,
)


def _merge_core(ps, cur, mw, mb):
    mwv = mw[...]
    s = jnp.zeros((B, TWO_L), jnp.float32)
    for i in range(4):
        pv = ps[i][...]
        m = jnp.tanh(pv[0] + pv[1])
        s = s + jnp.dot(m, mwv[i * L:(i + 1) * L, :],
                        preferred_element_type=jnp.float32)
    return jnp.tanh(s + mb[...] + cur[...])


def _merge_conv_body(p0, p1, p2, p3, cur, mw, mb, cw, cb,
                     cur_o, c0, c1, c2, c3):
    cur2 = _merge_core([p0, p1, p2, p3], cur, mw, mb)
    cur_o[...] = cur2
    cf = jnp.dot(cur2, cw[...], preferred_element_type=jnp.float32) + cb[...]
    c0[...] = cf[:, 0:L]
    c1[...] = cf[:, L:2 * L]
    c2[...] = cf[:, 2 * L:3 * L]
    c3[...] = cf[:, 3 * L:4 * L]


def _merge_final_body(p0, p1, p2, p3, cur, mw, mb, cur_o):
    cur_o[...] = _merge_core([p0, p1, p2, p3], cur, mw, mb)


def _pblk():
    return pl.BlockSpec((2, B, L), lambda i: (0, i, 0))


_merge_conv_call = pl.pallas_call(
    _merge_conv_body,
    grid=(GRID,),
    in_specs=[_pblk(), _pblk(), _pblk(), _pblk(),
              _rowblk(TWO_L), _full((FOUR_L, TWO_L)), _full((1, TWO_L)),
              _full((TWO_L, FOUR_L)), _full((1, FOUR_L))],
    out_specs=[_rowblk(TWO_L)] + [_rowblk(L)] * 4,
    out_shape=[jax.ShapeDtypeStruct((N, TWO_L), jnp.float32)]
    + [jax.ShapeDtypeStruct((N, L), jnp.float32)] * 4,
)

_merge_final_call = pl.pallas_call(
    _merge_final_body,
    grid=(GRID,),
    in_specs=[_pblk(), _pblk(), _pblk(), _pblk(),
              _rowblk(TWO_L), _full((FOUR_L, TWO_L)), _full((1, TWO_L))],
    out_specs=_rowblk(TWO_L),
    out_shape=jax.ShapeDtypeStruct((N, TWO_L), jnp.float32),
)


def kernel(node_feat, edge_index, edge_weight, all_embedding, wave_embedding,
           w_n2l_W, w_n2l_b, conv_W, conv_b, merge_W, merge_b):
    cur, c0, c1, c2, c3 = _pre_call(
        node_feat, all_embedding, wave_embedding, w_n2l_W,
        w_n2l_b.reshape(1, L), conv_W[0], conv_b[0].reshape(1, FOUR_L))
    def _blocks(x, pad_vals):
        pad = jnp.broadcast_to(pad_vals, (T, NW * CPW * K - E))
        x = jnp.concatenate([x, pad.astype(x.dtype)], axis=1)
        x = x.reshape(T, CPW, NW, K).transpose(0, 2, 1, 3)
        x = jnp.pad(x, ((0, 0), (0, 0), (0, CPW_PAD - CPW), (0, 0)))
        return x.reshape(T * NW * CPW_PAD, K)

    n_pad = NW * CPW * K - E
    idx_pad = jnp.arange(n_pad, dtype=jnp.int32)
    esrc = _blocks(edge_index[:, 0, :], idx_pad % N)
    edst = _blocks(edge_index[:, 1, :], N + idx_pad % (NP - N))
    ew = _blocks(edge_weight, jnp.zeros((n_pad,), jnp.float32))
    for lv in range(3):
        p0, p1, p2, p3 = _spmm_all_types(c0, c1, c2, c3, esrc, edst, ew)
        if lv < 2:
            cur, c0, c1, c2, c3 = _merge_conv_call(
                p0, p1, p2, p3, cur, merge_W[lv], merge_b[lv].reshape(1, TWO_L),
                conv_W[lv + 1], conv_b[lv + 1].reshape(1, FOUR_L))
        else:
            cur = _merge_final_call(
                p0, p1, p2, p3, cur, merge_W[2], merge_b[2].reshape(1, TWO_L))
    return cur


# speedup vs baseline: 1.1232x; 1.0194x over previous
import functools

import jax
import jax.numpy as jnp
from jax import lax
from jax.experimental import pallas as pl
from jax.experimental.pallas import tpu as pltpu
from jax.experimental.pallas import tpu_sc as plsc

N = 10000
NP = 10240
L = 128
TWO_L = 256
FOUR_L = 512
T = 4
E = 80000
K = 80
NW = 32
ROWS_PER_TILE = NP // 16
B = 2000
GRID = N // B


_mesh = plsc.VectorSubcoreMesh(core_axis_name="c", subcore_axis_name="s")


CPW = 33
CPW_PAD = 40


@functools.partial(
    pl.kernel,
    out_type=[jax.ShapeDtypeStruct((2, NP, L), jnp.float32) for _ in range(T)],
    mesh=_mesh,
    scratch_types=[
        pltpu.VMEM((CPW_PAD, K), jnp.int32),
        pltpu.VMEM((CPW_PAD, K), jnp.int32),
        pltpu.VMEM((CPW_PAD, K), jnp.float32),
        pltpu.VMEM((K, L), jnp.float32),
        pltpu.VMEM((K, L), jnp.float32),
        pltpu.VMEM((K, L), jnp.float32),
        pltpu.VMEM((16, L), jnp.float32),
        pltpu.VMEM_SHARED((NP, L), jnp.float32),
        pltpu.SemaphoreType.DMA,
        pltpu.SemaphoreType.DMA,
        pltpu.SemaphoreType.DMA,
        pltpu.SemaphoreType.DMA,
    ],
)
def _spmm_all_types(ch0, ch1, ch2, ch3, esrc, edst, ew, o0, o1, o2, o3,
                    sidx, didx, wv, rows0, rows1, rows2, zbuf, acc,
                    sem0, sem1, sem2, semi):
    cid = lax.axis_index("c")
    sid = lax.axis_index("s")
    wid = sid * 2 + cid
    row0 = sid * ROWS_PER_TILE

    z16 = jnp.zeros((16,), jnp.float32)

    def _zrow(r, carry):
        for c in range(8):
            zbuf[r, pl.ds(16 * c, 16)] = z16
        return carry

    lax.fori_loop(0, 16, _zrow, 0)

    chs = [ch0, ch1, ch2, ch3]
    outs = [o0, o1, o2, o3]

    for t in range(T):
        rb = (t * NW + wid) * CPW_PAD
        pltpu.async_copy(esrc.at[pl.ds(rb, CPW_PAD)], sidx, semi)
        pltpu.async_copy(edst.at[pl.ds(rb, CPW_PAD)], didx, semi)
        pltpu.async_copy(ew.at[pl.ds(rb, CPW_PAD)], wv, semi)
        pltpu.make_async_copy(esrc.at[pl.ds(rb, CPW_PAD)], sidx, semi).wait()
        pltpu.make_async_copy(edst.at[pl.ds(rb, CPW_PAD)], didx, semi).wait()
        pltpu.make_async_copy(ew.at[pl.ds(rb, CPW_PAD)], wv, semi).wait()

        ch = chs[t]

        def _gather(c, buf, sem):
            return pltpu.async_copy(ch.at[sidx.at[c]], buf, sem)

        def _wait(buf, sem):
            pltpu.make_async_copy(ch.at[sidx.at[0]], buf, sem).wait()

        _gather(0, rows0, sem0)
        _gather(1, rows1, sem1)

        def _zcopy(b, carry):
            pltpu.async_copy(zbuf, acc.at[pl.ds(row0 + 16 * b, 16)], semi)
            return carry

        lax.fori_loop(0, ROWS_PER_TILE // 16, _zcopy, 0)

        def _zwait(b, carry):
            pltpu.make_async_copy(zbuf, acc.at[pl.ds(row0, 16)], semi).wait()
            return carry

        lax.fori_loop(0, ROWS_PER_TILE // 16, _zwait, 0)
        plsc.subcore_barrier()

        def _scale_buf(c, buf):
            def _scale(g, c2):
                w16 = wv[c, pl.ds(g * 16, 16)]
                for ll in range(16):
                    j = g * 16 + ll
                    wsp = w16[ll]
                    for cc in range(8):
                        sl = pl.ds(16 * cc, 16)
                        buf[j, sl] = buf[j, sl] * wsp
                return c2

            lax.fori_loop(0, K // 16, _scale, 0)

        def _wait_scatter(buf, sem):
            pltpu.make_async_copy(buf, acc.at[didx.at[0]], sem).wait()

        ring = [(rows0, sem0), (rows1, sem1), (rows2, sem2)]

        def _triple(q, carry):
            for phase in range(3):
                buf, sem = ring[phase]
                nbuf, nsem = ring[(phase + 2) % 3]
                c = 3 * q + phase
                _wait(buf, sem)
                _scale_buf(c, buf)
                pltpu.async_copy(buf, acc.at[didx.at[c]], sem, add=True)
                if phase == 0:
                    @pl.when(q > 0)
                    def _():
                        _wait_scatter(nbuf, nsem)

                    _gather(c + 2, nbuf, nsem)
                else:
                    _wait_scatter(nbuf, nsem)

                    @pl.when(q < CPW // 3 - 1)
                    def _():
                        _gather(c + 2, nbuf, nsem)
            return carry

        lax.fori_loop(0, CPW // 3, _triple, 0)
        _wait_scatter(rows2, sem2)
        plsc.subcore_barrier()

        for b5 in range(5):
            sl = pl.ds(row0 + 128 * b5, 128)
            pltpu.async_copy(acc.at[sl], outs[t].at[cid, sl], semi)
        for b5 in range(5):
            sl = pl.ds(row0 + 128 * b5, 128)
            pltpu.make_async_copy(acc.at[sl], outs[t].at[cid, sl], semi).wait()


def _full(shape):
    return pl.BlockSpec(shape, lambda i, _s=shape: tuple(0 for _ in _s))


def _rowblk(w):
    return pl.BlockSpec((B, w), lambda i: (i, 0))


def _pre_body(nf, ae, wav, w1, b1, cw, cb, cur_o, c0, c1, c2, c3):
    ws = jnp.sum(wav[...], axis=0, keepdims=True)
    left = jnp.tanh(
        jnp.dot(nf[...], w1[...], preferred_element_type=jnp.float32)
        + b1[...] + ae[...])
    right = jnp.broadcast_to(jnp.tanh(ws), (B, L))
    cur = jnp.concatenate([left, right], axis=1)
    cur_o[...] = cur
    cf = jnp.dot(cur, cw[...], preferred_element_type=jnp.float32) + cb[...]
    c0[...] = cf[:, 0:L]
    c1[...] = cf[:, L:2 * L]
    c2[...] = cf[:, 2 * L:3 * L]
    c3[...] = cf[:, 3 * L:4 * L]


_pre_call = pl.pallas_call(
    _pre_body,
    grid=(GRID,),
    in_specs=[_rowblk(L), _rowblk(L), _full((512, L)), _full((L, L)),
              _full((1, L)), _full((TWO_L, FOUR_L)), _full((1, FOUR_L))],
    out_specs=[_rowblk(TWO_L)] + [_rowblk(L)] * 4,
    out_shape=[jax.ShapeDtypeStruct((N, TWO_L), jnp.float32)]
    + [jax.ShapeDtypeStruct((N, L), jnp.float32)] * 4,
)


def _merge_core(ps, cur, mw, mb):
    mwv = mw[...]
    s = jnp.zeros((B, TWO_L), jnp.float32)
    for i in range(4):
        pv = ps[i][...]
        m = jnp.tanh(pv[0] + pv[1])
        s = s + jnp.dot(m, mwv[i * L:(i + 1) * L, :],
                        preferred_element_type=jnp.float32)
    return jnp.tanh(s + mb[...] + cur[...])


def _merge_conv_body(p0, p1, p2, p3, cur, mw, mb, cw, cb,
                     cur_o, c0, c1, c2, c3):
    cur2 = _merge_core([p0, p1, p2, p3], cur, mw, mb)
    cur_o[...] = cur2
    cf = jnp.dot(cur2, cw[...], preferred_element_type=jnp.float32) + cb[...]
    c0[...] = cf[:, 0:L]
    c1[...] = cf[:, L:2 * L]
    c2[...] = cf[:, 2 * L:3 * L]
    c3[...] = cf[:, 3 * L:4 * L]


def _merge_final_body(p0, p1, p2, p3, cur, mw, mb, cur_o):
    cur_o[...] = _merge_core([p0, p1, p2, p3], cur, mw, mb)


def _pblk():
    return pl.BlockSpec((2, B, L), lambda i: (0, i, 0))


_merge_conv_call = pl.pallas_call(
    _merge_conv_body,
    grid=(GRID,),
    in_specs=[_pblk(), _pblk(), _pblk(), _pblk(),
              _rowblk(TWO_L), _full((FOUR_L, TWO_L)), _full((1, TWO_L)),
              _full((TWO_L, FOUR_L)), _full((1, FOUR_L))],
    out_specs=[_rowblk(TWO_L)] + [_rowblk(L)] * 4,
    out_shape=[jax.ShapeDtypeStruct((N, TWO_L), jnp.float32)]
    + [jax.ShapeDtypeStruct((N, L), jnp.float32)] * 4,
)

_merge_final_call = pl.pallas_call(
    _merge_final_body,
    grid=(GRID,),
    in_specs=[_pblk(), _pblk(), _pblk(), _pblk(),
              _rowblk(TWO_L), _full((FOUR_L, TWO_L)), _full((1, TWO_L))],
    out_specs=_rowblk(TWO_L),
    out_shape=jax.ShapeDtypeStruct((N, TWO_L), jnp.float32),
)


def kernel(node_feat, edge_index, edge_weight, all_embedding, wave_embedding,
           w_n2l_W, w_n2l_b, conv_W, conv_b, merge_W, merge_b):
    cur, c0, c1, c2, c3 = _pre_call(
        node_feat, all_embedding, wave_embedding, w_n2l_W,
        w_n2l_b.reshape(1, L), conv_W[0], conv_b[0].reshape(1, FOUR_L))
    def _blocks(x, pad_vals):
        pad = jnp.broadcast_to(pad_vals, (T, NW * CPW * K - E))
        x = jnp.concatenate([x, pad.astype(x.dtype)], axis=1)
        x = x.reshape(T, CPW, NW, K).transpose(0, 2, 1, 3)
        x = jnp.pad(x, ((0, 0), (0, 0), (0, CPW_PAD - CPW), (0, 0)))
        return x.reshape(T * NW * CPW_PAD, K)

    n_pad = NW * CPW * K - E
    idx_pad = jnp.arange(n_pad, dtype=jnp.int32)
    esrc = _blocks(edge_index[:, 0, :], idx_pad % N)
    edst = _blocks(edge_index[:, 1, :], N + idx_pad % (NP - N))
    ew = _blocks(edge_weight, jnp.zeros((n_pad,), jnp.float32))
    for lv in range(3):
        p0, p1, p2, p3 = _spmm_all_types(c0, c1, c2, c3, esrc, edst, ew)
        if lv < 2:
            cur, c0, c1, c2, c3 = _merge_conv_call(
                p0, p1, p2, p3, cur, merge_W[lv], merge_b[lv].reshape(1, TWO_L),
                conv_W[lv + 1], conv_b[lv + 1].reshape(1, FOUR_L))
        else:
            cur = _merge_final_call(
                p0, p1, p2, p3, cur, merge_W[2], merge_b[2].reshape(1, TWO_L))
    return cur
